# baseline (speedup 1.0000x reference)
import jax
import jax.numpy as jnp
from jax.experimental import pallas as pl
from jax.experimental.pallas import tpu as pltpu

_N = 4096
_D = 2000
_H1 = 512
_H2 = 128
_HD = 512

_BK = 256
_BI = 1024
_BM = 512


def _leaky(v):
    return jnp.where(v > 0, v, 0.01 * v)


def _dot(a, b):
    return jnp.dot(a, b, preferred_element_type=jnp.float32)


def _enc_kernel(x_ref, adj_ref, gc1_ref, g2_ref, fc1w_ref, fc1b_ref,
                mu_ref, lv_ref, h_ref, stats_ref, s1_scr, s2_scr):
    t = pl.program_id(0)
    nk = _N // _BK

    @pl.when(t < nk)
    def _phase1():
        s1_scr[pl.ds(t * _BK, _BK), :] = _dot(x_ref[...], gc1_ref[...])

    @pl.when(jnp.logical_and(t >= nk, t < 2 * nk))
    def _phase2():
        h1 = _leaky(_dot(adj_ref[...], s1_scr[...]))
        s2_scr[pl.ds((t - nk) * _BK, _BK), :] = _dot(h1, g2_ref[...])

    @pl.when(t >= 2 * nk)
    def _phase3():
        ml = _leaky(_dot(adj_ref[...], s2_scr[...]))
        mu = ml[:, :_H2]
        mu_ref[...] = mu
        lv_ref[...] = ml[:, _H2:]
        h = _dot(mu, fc1w_ref[...]) + fc1b_ref[...]
        h_ref[...] = h
        s = jnp.concatenate(
            [jnp.sum(h, axis=0, keepdims=True),
             jnp.sum(h * h, axis=0, keepdims=True)], axis=0)

        @pl.when(t == 2 * nk)
        def _init():
            stats_ref[...] = s

        @pl.when(t > 2 * nk)
        def _acc():
            stats_ref[...] += s


def _dec_kernel(z_ref, zi_ref, h_ref, stats_ref, gamma_ref, beta_ref,
                thw_ref, thb_ref, mw_ref, mb_ref, piw_ref, pib_ref,
                rec_ref, out_ref, pi_ref, th_ref, mr_ref):
    rec_ref[...] = jax.lax.dot_general(
        zi_ref[...], z_ref[...], (((1,), (1,)), ((), ())),
        preferred_element_type=jnp.float32)
    s = stats_ref[...]
    bm = s[0:1, :] * (1.0 / _N)
    bv = s[1:2, :] * (1.0 / _N) - bm * bm
    inv = jax.lax.rsqrt(bv + 1e-5)
    out = _leaky((h_ref[...] - bm) * (inv * gamma_ref[...]) + beta_ref[...])
    out_ref[...] = out
    th = _dot(out, thw_ref[...]) + thb_ref[...]
    th_ref[...] = jnp.clip(jax.nn.softplus(th), 1e-5, 1e6)
    mn = _dot(out, mw_ref[...]) + mb_ref[...]
    mr_ref[...] = jnp.clip(jnp.exp(mn), 1e-5, 1e6)
    pi_ref[...] = jax.nn.sigmoid(mn * piw_ref[...] + pib_ref[...])


def kernel(x, adj, gc1_w, gc2_w, gc2s_w, fc1_w, fc1_b, fc1_gamma, fc1_beta,
           theta_w, theta_b, mean_w, mean_b, pi_w, pi_b):
    f32 = jnp.float32

    g2 = jnp.concatenate([gc2_w, gc2s_w], axis=1)
    nk = _N // _BK
    fc1_b2 = fc1_b.reshape(1, _HD)
    mu, logvar, h, stats = pl.pallas_call(
        _enc_kernel,
        grid=(3 * nk,),
        in_specs=[
            pl.BlockSpec((_BK, _D), lambda t: (jnp.minimum(t, nk - 1), 0)),
            pl.BlockSpec(
                (_BK, _N),
                lambda t: (jnp.where(t >= 2 * nk, t - 2 * nk,
                                     jnp.maximum(t - nk, 0)), 0)),
            pl.BlockSpec((_D, _H1), lambda t: (0, 0)),
            pl.BlockSpec((_H1, 2 * _H2), lambda t: (0, 0)),
            pl.BlockSpec((_H2, _HD), lambda t: (0, 0)),
            pl.BlockSpec((1, _HD), lambda t: (0, 0)),
        ],
        out_specs=[
            pl.BlockSpec((_BK, _H2),
                         lambda t: (jnp.maximum(t - 2 * nk, 0), 0)),
            pl.BlockSpec((_BK, _H2),
                         lambda t: (jnp.maximum(t - 2 * nk, 0), 0)),
            pl.BlockSpec((_BK, _HD),
                         lambda t: (jnp.maximum(t - 2 * nk, 0), 0)),
            pl.BlockSpec((2, _HD), lambda t: (0, 0)),
        ],
        out_shape=[
            jax.ShapeDtypeStruct((_N, _H2), f32),
            jax.ShapeDtypeStruct((_N, _H2), f32),
            jax.ShapeDtypeStruct((_N, _HD), f32),
            jax.ShapeDtypeStruct((2, _HD), f32),
        ],
        scratch_shapes=[
            pltpu.VMEM((_N, _H1), f32),
            pltpu.VMEM((_N, 2 * _H2), f32),
        ],
        compiler_params=pltpu.CompilerParams(
            dimension_semantics=("arbitrary",)),
    )(x, adj, gc1_w, g2, fc1_w, fc1_b2)
    z = mu

    nblk = _N // _BM
    adj_rec, output, pi_res, theta_res, mean_res = pl.pallas_call(
        _dec_kernel,
        grid=(nblk,),
        in_specs=[
            pl.BlockSpec((_N, _H2), lambda i: (0, 0)),
            pl.BlockSpec((_BM, _H2), lambda i: (i, 0)),
            pl.BlockSpec((_BM, _HD), lambda i: (i, 0)),
            pl.BlockSpec((2, _HD), lambda i: (0, 0)),
            pl.BlockSpec((1, _HD), lambda i: (0, 0)),
            pl.BlockSpec((1, _HD), lambda i: (0, 0)),
            pl.BlockSpec((_HD, _D), lambda i: (0, 0)),
            pl.BlockSpec((1, _D), lambda i: (0, 0)),
            pl.BlockSpec((_HD, _D), lambda i: (0, 0)),
            pl.BlockSpec((1, _D), lambda i: (0, 0)),
            pl.BlockSpec((1, _D), lambda i: (0, 0)),
            pl.BlockSpec((1, _D), lambda i: (0, 0)),
        ],
        out_specs=[
            pl.BlockSpec((_BM, _N), lambda i: (i, 0)),
            pl.BlockSpec((_BM, _HD), lambda i: (i, 0)),
            pl.BlockSpec((_BM, _D), lambda i: (i, 0)),
            pl.BlockSpec((_BM, _D), lambda i: (i, 0)),
            pl.BlockSpec((_BM, _D), lambda i: (i, 0)),
        ],
        out_shape=[
            jax.ShapeDtypeStruct((_N, _N), f32),
            jax.ShapeDtypeStruct((_N, _HD), f32),
            jax.ShapeDtypeStruct((_N, _D), f32),
            jax.ShapeDtypeStruct((_N, _D), f32),
            jax.ShapeDtypeStruct((_N, _D), f32),
        ],
        compiler_params=pltpu.CompilerParams(
            dimension_semantics=("arbitrary",)),
    )(z, z, h, stats, fc1_gamma.reshape(1, _HD), fc1_beta.reshape(1, _HD),
      theta_w, theta_b.reshape(1, _D), mean_w, mean_b.reshape(1, _D),
      pi_w.reshape(1, _D), pi_b.reshape(1, _D))

    return (adj_rec, mu, logvar, z, output, pi_res, theta_res, mean_res)
